# Initial kernel scaffold; baseline (speedup 1.0000x reference)
import jax
import jax.numpy as jnp
from jax.experimental import pallas as pl


def kernel(x, pe, W1, b1, W2, b2):
    raise NotImplementedError("write your pallas kernel here")


# speedup vs baseline: 1.4000x; 1.4000x over previous
import functools

import jax
import jax.numpy as jnp
from jax import lax
from jax.experimental import pallas as pl
from jax.experimental.pallas import tpu as pltpu
from jax.experimental.pallas import tpu_sc as plsc

T_ROWS = 1000
D_IN = 128
D_OUT = 512
B = 16384

_info = plsc.get_sparse_core_info()
NC, NS = _info.num_cores, _info.num_subcores
NW = NC * NS
BPW = B // NW
C = 128
NCHUNK = BPW // C


def _table_body(pe_ref, w1_ref, b1_ref, w2_ref, b2_ref, out_ref):
    h = jnp.dot(pe_ref[...], w1_ref[...], preferred_element_type=jnp.float32)
    h = h + b1_ref[...]
    h = h * jax.nn.sigmoid(h)
    out_ref[...] = (
        jnp.dot(h, w2_ref[...], preferred_element_type=jnp.float32) + b2_ref[...]
    )


def _compute_table(pe, W1, b1, W2, b2):
    return pl.pallas_call(
        _table_body,
        out_shape=jax.ShapeDtypeStruct((T_ROWS, D_OUT), jnp.float32),
    )(pe, W1, b1.reshape(1, D_OUT), W2, b2.reshape(1, D_OUT))


_mesh = plsc.VectorSubcoreMesh(core_axis_name="c", subcore_axis_name="s")


@functools.partial(
    pl.kernel,
    mesh=_mesh,
    out_type=jax.ShapeDtypeStruct((B, D_OUT), jnp.float32),
    scratch_types=[
        pltpu.VMEM((NCHUNK, C), jnp.int32),
        pltpu.VMEM((C, D_OUT), jnp.float32),
        pltpu.SemaphoreType.DMA,
    ],
)
def _sc_gather(table_hbm, idx_hbm, out_hbm, idx_v, rows_v, sem):
    wid = lax.axis_index("s") * NC + lax.axis_index("c")
    pltpu.sync_copy(idx_hbm.at[wid], idx_v)
    for c in range(NCHUNK):
        pltpu.async_copy(table_hbm.at[idx_v.at[c]], rows_v, sem).wait()
        pltpu.sync_copy(rows_v, out_hbm.at[pl.ds(wid * BPW + c * C, C)])


def kernel(x, pe, W1, b1, W2, b2):
    table = _compute_table(pe, W1, b1, W2, b2)
    idx = x.astype(jnp.int32).reshape(NW, NCHUNK, C)
    return _sc_gather(table, idx)


# speedup vs baseline: 1.4064x; 1.0046x over previous
import functools

import jax
import jax.numpy as jnp
from jax import lax
from jax.experimental import pallas as pl
from jax.experimental.pallas import tpu as pltpu
from jax.experimental.pallas import tpu_sc as plsc

T_ROWS = 1000
D_IN = 128
D_OUT = 512
B = 16384

_info = plsc.get_sparse_core_info()
NC, NS = _info.num_cores, _info.num_subcores
NW = NC * NS
BPW = B // NW
C = 64
NCHUNK = BPW // C
NBUF = 3


def _table_body(pe_ref, w1_ref, b1_ref, w2_ref, b2_ref, out_ref):
    h = jnp.dot(pe_ref[...], w1_ref[...], preferred_element_type=jnp.float32)
    h = h + b1_ref[...]
    h = h * jax.nn.sigmoid(h)
    out_ref[...] = (
        jnp.dot(h, w2_ref[...], preferred_element_type=jnp.float32) + b2_ref[...]
    )


def _compute_table(pe, W1, b1, W2, b2):
    return pl.pallas_call(
        _table_body,
        out_shape=jax.ShapeDtypeStruct((T_ROWS, D_OUT), jnp.float32),
    )(pe, W1, b1.reshape(1, D_OUT), W2, b2.reshape(1, D_OUT))


_mesh = plsc.VectorSubcoreMesh(core_axis_name="c", subcore_axis_name="s")


@functools.partial(
    pl.kernel,
    mesh=_mesh,
    out_type=jax.ShapeDtypeStruct((B, D_OUT), jnp.float32),
    scratch_types=[
        pltpu.VMEM((NCHUNK, C), jnp.int32),
        *[pltpu.VMEM((C, D_OUT), jnp.float32) for _ in range(NBUF)],
        pltpu.SemaphoreType.DMA,
        pltpu.SemaphoreType.DMA,
    ],
)
def _sc_gather(table_hbm, idx_hbm, out_hbm, idx_v, *rest):
    bufs, (gsem, ssem) = rest[:NBUF], rest[NBUF:]
    wid = lax.axis_index("s") * NC + lax.axis_index("c")
    base = wid * BPW
    pltpu.sync_copy(idx_hbm.at[wid], idx_v)
    g = [None] * NCHUNK
    s = [None] * NCHUNK
    for c in range(min(2, NCHUNK)):
        g[c] = pltpu.async_copy(table_hbm.at[idx_v.at[c]], bufs[c % NBUF], gsem)
    for c in range(NCHUNK):
        g[c].wait()
        n = c + 2
        if n < NCHUNK:
            if n - NBUF >= 0:
                s[n - NBUF].wait()
            g[n] = pltpu.async_copy(table_hbm.at[idx_v.at[n]], bufs[n % NBUF], gsem)
        s[c] = pltpu.async_copy(
            bufs[c % NBUF], out_hbm.at[pl.ds(base + c * C, C)], ssem
        )
    for c in range(max(0, NCHUNK - NBUF), NCHUNK):
        s[c].wait()


def kernel(x, pe, W1, b1, W2, b2):
    table = _compute_table(pe, W1, b1, W2, b2)
    idx = x.astype(jnp.int32).reshape(NW, NCHUNK, C)
    return _sc_gather(table, idx)
